# baseline (speedup 1.0000x reference)
import functools

import jax
import jax.numpy as jnp
from jax import lax
from jax.experimental import pallas as pl
from jax.experimental.pallas import tpu as pltpu
from jax.experimental.pallas import tpu_sc as plsc

S = 2048
B = 4
D = 128
N = S * B
NW = 32
SPW = S // NW
LANES = 16


def _emb_body(xt_hbm, pe_hbm, table_hbm, out_hbm, idx_v, idxo_v, rows_v,
              pe_v, sems):
    wid = lax.axis_index("s") * 2 + lax.axis_index("c")
    s0 = wid * SPW

    gathers = []
    for b in range(B):
        pltpu.sync_copy(xt_hbm.at[pl.ds(b * S + s0, SPW)], idx_v.at[b])
        gathers.append(pltpu.async_copy(
            table_hbm.at[idx_v.at[b]],
            rows_v.at[b],
            sems.at[b],
        ))
    pltpu.sync_copy(pe_hbm.at[pl.ds(s0, SPW)], pe_v)

    lane = lax.broadcasted_iota(jnp.int32, (LANES,), 0)
    for b in range(B):
        for m in range(SPW // LANES):
            idxo_v[b, pl.ds(m * LANES, LANES)] = (
                (s0 * B + b + 4 * m * LANES) + lane * B)

    outs = []
    for c in range(2):
        gathers[2 * c].wait()
        gathers[2 * c + 1].wait()

        @plsc.parallel_loop(0, SPW * (D // LANES), unroll=2)
        def add_body(t):
            i = t >> 3
            sl = pl.ds((t & 7) * LANES, LANES)
            p = pe_v[i, 0, sl]
            rows_v[2 * c, i, sl] = rows_v[2 * c, i, sl] + p
            rows_v[2 * c + 1, i, sl] = rows_v[2 * c + 1, i, sl] + p

        for b in (2 * c, 2 * c + 1):
            outs.append(pltpu.async_copy(
                rows_v.at[b],
                out_hbm.at[idxo_v.at[b]],
                sems.at[B + b],
            ))
    for o in outs:
        o.wait()


@jax.jit
def _emb(xt, pe, table):
    mesh = plsc.VectorSubcoreMesh(core_axis_name="c", subcore_axis_name="s")
    f = functools.partial(
        pl.kernel,
        mesh=mesh,
        out_type=jax.ShapeDtypeStruct((N, D), jnp.float32),
        scratch_types=[
            pltpu.VMEM((B, SPW), jnp.int32),
            pltpu.VMEM((B, SPW), jnp.int32),
            pltpu.VMEM((B, SPW, D), jnp.float32),
            pltpu.VMEM((SPW, 1, D), jnp.float32),
            pltpu.SemaphoreType.DMA((2 * B,)),
        ],
    )(_emb_body)
    return f(xt, pe, table)


def kernel(x, table, pe):
    xt = x.T.reshape(N)
    out = _emb(xt, pe, table)
    return out.reshape(S, B, D)


# speedup vs baseline: 1.0251x; 1.0251x over previous
import functools

import jax
import jax.numpy as jnp
from jax import lax
from jax.experimental import pallas as pl
from jax.experimental.pallas import tpu as pltpu
from jax.experimental.pallas import tpu_sc as plsc

S = 2048
B = 4
D = 128
N = S * B
NW = 32
RPW = N // NW
SPW = S // NW
LANES = 16
NCHUNK = 2
RPC = RPW // NCHUNK
SPC = SPW // NCHUNK


def _emb_body(x_hbm, pe_hbm, table_hbm, out_hbm, idx_v, rows_v, pe_v, sems):
    wid = lax.axis_index("s") * 2 + lax.axis_index("c")
    base = wid * RPW
    s0 = wid * SPW

    pltpu.sync_copy(x_hbm.at[pl.ds(base, RPW)], idx_v)
    gathers = []
    for c in range(NCHUNK):
        gathers.append(pltpu.async_copy(
            table_hbm.at[idx_v.at[pl.ds(c * RPC, RPC)]],
            rows_v.at[pl.ds(c * RPC, RPC)],
            sems.at[c],
        ))
    pltpu.sync_copy(pe_hbm.at[pl.ds(s0, SPW)], pe_v)

    outs = []
    for c in range(NCHUNK):
        gathers[c].wait()

        @plsc.parallel_loop(c * SPC * 8, (c + 1) * SPC * 8, unroll=1)
        def add_body(t):
            r = t >> 3
            sl = pl.ds((t & 7) * LANES, LANES)
            row = r * B
            p = pe_v[r, 0, sl]
            for b in range(B):
                rows_v[row + b, sl] = rows_v[row + b, sl] + p

        outs.append(pltpu.async_copy(
            rows_v.at[pl.ds(c * RPC, RPC)],
            out_hbm.at[pl.ds(base + c * RPC, RPC)],
            sems.at[NCHUNK + c],
        ))
    for o in outs:
        o.wait()


@jax.jit
def _emb(x1, pe, table):
    mesh = plsc.VectorSubcoreMesh(core_axis_name="c", subcore_axis_name="s")
    f = functools.partial(
        pl.kernel,
        mesh=mesh,
        out_type=jax.ShapeDtypeStruct((N, D), jnp.float32),
        scratch_types=[
            pltpu.VMEM((RPW,), jnp.int32),
            pltpu.VMEM((RPW, D), jnp.float32),
            pltpu.VMEM((SPW, 1, D), jnp.float32),
            pltpu.SemaphoreType.DMA((2 * NCHUNK,)),
        ],
    )(_emb_body)
    return f(x1, pe, table)


def kernel(x, table, pe):
    x1 = x.reshape(N)
    out = _emb(x1, pe, table)
    return out.reshape(S, B, D)


# speedup vs baseline: 1.0415x; 1.0160x over previous
import functools

import jax
import jax.numpy as jnp
from jax import lax
from jax.experimental import pallas as pl
from jax.experimental.pallas import tpu as pltpu
from jax.experimental.pallas import tpu_sc as plsc

S = 2048
B = 4
D = 128
N = S * B
NW = 32
RPW = N // NW
SPW = S // NW
LANES = 16
NCHUNK = 2
RPC = RPW // NCHUNK
SPC = SPW // NCHUNK


def _emb_body(x_hbm, pe_hbm, table_hbm, out_hbm, idx_v, rows_v, pe_v, sems):
    wid = lax.axis_index("s") * 2 + lax.axis_index("c")
    base = wid * RPW
    s0 = wid * SPW

    pltpu.sync_copy(x_hbm.at[pl.ds(base, RPW)], idx_v)
    gathers = []
    for c in range(NCHUNK):
        gathers.append(pltpu.async_copy(
            table_hbm.at[idx_v.at[pl.ds(c * RPC, RPC)]],
            rows_v.at[pl.ds(c * RPC, RPC)],
            sems.at[c],
        ))
    pltpu.sync_copy(pe_hbm.at[pl.ds(s0, SPW)], pe_v)

    outs = []
    for c in range(NCHUNK):
        gathers[c].wait()

        @plsc.parallel_loop(c * SPC * 8, (c + 1) * SPC * 8, unroll=2)
        def add_body(t):
            r = t >> 3
            sl = pl.ds((t & 7) * LANES, LANES)
            row = r * B
            p = pe_v[r, 0, sl]
            for b in range(B):
                rows_v[row + b, sl] = rows_v[row + b, sl] + p

        outs.append(pltpu.async_copy(
            rows_v.at[pl.ds(c * RPC, RPC)],
            out_hbm.at[pl.ds(base + c * RPC, RPC)],
            sems.at[NCHUNK + c],
        ))
    for o in outs:
        o.wait()


@jax.jit
def _emb(x1, pe, table):
    mesh = plsc.VectorSubcoreMesh(core_axis_name="c", subcore_axis_name="s")
    f = functools.partial(
        pl.kernel,
        mesh=mesh,
        out_type=jax.ShapeDtypeStruct((N, D), jnp.float32),
        scratch_types=[
            pltpu.VMEM((RPW,), jnp.int32),
            pltpu.VMEM((RPW, D), jnp.float32),
            pltpu.VMEM((SPW, 1, D), jnp.float32),
            pltpu.SemaphoreType.DMA((2 * NCHUNK,)),
        ],
    )(_emb_body)
    return f(x1, pe, table)


def kernel(x, table, pe):
    x1 = x.reshape(N)
    out = _emb(x1, pe, table)
    return out.reshape(S, B, D)
